# baseline (speedup 1.0000x reference)
import dataclasses
import functools
import math

import jax
import jax.numpy as jnp
from jax import lax
from jax.experimental import pallas as pl
from jax.experimental.pallas import tpu as pltpu
from jax.experimental.pallas import tpu_sc as plsc

N = 10000
D = 128
N_T = 10048
N_PAD = 10240
HR = N_PAD // 128
G = 56
BATCH = 8
NC = 2
NS = 16
NW = NC * NS
STRIPE = N_PAD // NS


def _sc_compiler_params():
    cp = pltpu.CompilerParams()
    if "needs_layout_passes" in pltpu.CompilerParams.__dataclass_fields__:
        cp = dataclasses.replace(cp, needs_layout_passes=False)
    return cp


def _tc_tables_body(x_ref, w_ref, root_ref, b_ref, tab_ref, xrb_ref):
    x = x_ref[...]
    w0 = w_ref[0]
    wd = w_ref[1] - w_ref[0]
    tab_ref[:N, :D] = jnp.dot(x, w0, preferred_element_type=jnp.float32)
    tab_ref[:N, D:] = jnp.dot(x, wd, preferred_element_type=jnp.float32)
    tab_ref[N:, :] = jnp.zeros((N_T - N, 2 * D), jnp.float32)
    xrb_ref[...] = (jnp.dot(x, root_ref[...], preferred_element_type=jnp.float32)
                    + b_ref[...])


def _combine(acc_ref, d0_ref, d1_ref, xrb_ref):
    num = acc_ref[0, :N, :] + acc_ref[1, :N, :]
    rec = 1.0 / jnp.clip(d0_ref[...] + d1_ref[...], 1.0, None)
    return num * rec + xrb_ref[...]


def _tc_mid_body(acc_ref, d0_ref, d1_ref, xrb_ref, w_ref, root_ref, b_ref,
                 tab_ref, xrb2_ref):
    h = _combine(acc_ref, d0_ref, d1_ref, xrb_ref)
    h = jnp.where(h > 0, h, jnp.exp(h) - 1.0)
    w0 = w_ref[0]
    wd = w_ref[1] - w_ref[0]
    tab_ref[:N, :D] = jnp.dot(h, w0, preferred_element_type=jnp.float32)
    tab_ref[:N, D:] = jnp.dot(h, wd, preferred_element_type=jnp.float32)
    tab_ref[N:, :] = jnp.zeros((N_T - N, 2 * D), jnp.float32)
    xrb2_ref[...] = (jnp.dot(h, root_ref[...], preferred_element_type=jnp.float32)
                     + b_ref[...])


def _tc_out_body(acc_ref, d0_ref, d1_ref, xrb_ref, out_ref):
    z = _combine(acc_ref, d0_ref, d1_ref, xrb_ref)
    m = jnp.max(z, axis=1, keepdims=True)
    zz = z - m
    lse = jnp.log(jnp.sum(jnp.exp(zz), axis=1, keepdims=True))
    out_ref[...] = zz - lse


_tc_tables = pl.pallas_call(
    _tc_tables_body,
    out_shape=(jax.ShapeDtypeStruct((N_T, 2 * D), jnp.float32),
               jax.ShapeDtypeStruct((N, D), jnp.float32)),
)

_tc_mid = pl.pallas_call(
    _tc_mid_body,
    out_shape=(jax.ShapeDtypeStruct((N_T, 2 * D), jnp.float32),
               jax.ShapeDtypeStruct((N, D), jnp.float32)),
)

_tc_out = pl.pallas_call(
    _tc_out_body,
    out_shape=jax.ShapeDtypeStruct((N, D), jnp.float32),
)


def _make_sc_deg(chunks_per_tile):
    mesh = plsc.VectorSubcoreMesh(core_axis_name="c", subcore_axis_name="s")

    @functools.partial(
        pl.kernel,
        out_type=jax.ShapeDtypeStruct((NC, HR, 128), jnp.float32),
        mesh=mesh,
        compiler_params=_sc_compiler_params(),
        scratch_types=[
            pltpu.VMEM((1, 128), jnp.int32),
            pltpu.VMEM((HR, 128), jnp.float32),
            pltpu.VMEM((1, HR), jnp.int32),
            pltpu.VMEM_SHARED((HR, 128), jnp.float32),
        ],
    )
    def sc_deg(dst_hbm, out_hbm, dst_v, hist_v, idx_v, hist_sh):
        cid = lax.axis_index("c")
        sid = lax.axis_index("s")
        wid = cid * NS + sid

        @pl.loop(0, HR)
        def _(r):
            for k in range(128 // 16):
                hist_v[r, pl.ds(k * 16, 16)] = jnp.zeros((16,), jnp.float32)

        for k in range(HR // 16):
            idx_v[0, pl.ds(k * 16, 16)] = lax.iota(jnp.int32, 16) + (k * 16)

        @pl.when(sid < HR // 8)
        def _():
            pltpu.sync_copy(hist_v.at[pl.ds(0, 8)],
                            hist_sh.at[pl.ds(sid * 8, 8)])
        plsc.subcore_barrier()

        ones16 = jnp.ones((16,), jnp.float32)

        @pl.loop(0, chunks_per_tile)
        def _(j):
            row = wid * chunks_per_tile + j
            pltpu.sync_copy(dst_hbm.at[pl.ds(row, 1)], dst_v)
            for t in range(128 // 16):
                d = dst_v[0, pl.ds(t * 16, 16)]
                plsc.addupdate_scatter(
                    hist_v,
                    [lax.shift_right_logical(d, 7), lax.bitwise_and(d, 127)],
                    ones16)

        pltpu.sync_copy(hist_v, hist_sh.at[idx_v.at[0]], add=True)
        plsc.subcore_barrier()

        @pl.when(sid < HR // 8)
        def _():
            pltpu.sync_copy(hist_sh.at[pl.ds(sid * 8, 8)],
                            out_hbm.at[cid, pl.ds(sid * 8, 8)])

    return sc_deg


def _make_sc_edge(units_per_tile):
    assert units_per_tile % 2 == 0 and units_per_tile % BATCH == 0
    batches_per_tile = units_per_tile // BATCH
    mesh = plsc.VectorSubcoreMesh(core_axis_name="c", subcore_axis_name="s")

    @functools.partial(
        pl.kernel,
        out_type=jax.ShapeDtypeStruct((NC, N_PAD, D), jnp.float32),
        mesh=mesh,
        compiler_params=_sc_compiler_params(),
        scratch_types=[
            pltpu.VMEM((2, BATCH, G), jnp.int32),
            pltpu.VMEM((2, BATCH, G), jnp.int32),
            pltpu.VMEM((G // 8, 128), jnp.float32),
            pltpu.VMEM((G // 8, 128), jnp.float32),
            pltpu.VMEM((G, 2 * D), jnp.float32),
            pltpu.VMEM((G, 2 * D), jnp.float32),
            pltpu.VMEM((G, D), jnp.float32),
            pltpu.VMEM((G, D), jnp.float32),
            pltpu.VMEM_SHARED((N_PAD, D), jnp.float32),
            pltpu.SemaphoreType.DMA,
            pltpu.SemaphoreType.DMA,
        ],
    )
    def sc_edge(tab_hbm, src_hbm, dst_hbm, attrw_hbm, out_hbm,
                src_b, dst_b, aw0, aw1, rows0, rows1, msg0, msg1,
                acc_sh, sem_g, sem_s):
        cid = lax.axis_index("c")
        sid = lax.axis_index("s")
        wid = cid * NS + sid

        @pl.loop(0, G)
        def _(i):
            for k in range(D // 16):
                msg0[i, pl.ds(k * 16, 16)] = jnp.zeros((16,), jnp.float32)

        @pl.loop(0, STRIPE // 8)
        def _(j):
            pltpu.sync_copy(msg0.at[pl.ds(0, 8)],
                            acc_sh.at[pl.ds(sid * STRIPE + j * 8, 8)])

        plsc.subcore_barrier()

        def load_batch(b):
            row = wid * batches_per_tile + b
            slot = lax.rem(b, 2)
            pltpu.sync_copy(src_hbm.at[row], src_b.at[slot])
            pltpu.sync_copy(dst_hbm.at[row], dst_b.at[slot])

        def gather_descs(u, r_buf, a_buf):
            ub = lax.rem(lax.div(u, BATCH), 2)
            um = lax.rem(u, BATCH)
            idx = src_b.at[ub, um]
            return (pltpu.make_async_copy(tab_hbm.at[idx], r_buf, sem_g),
                    pltpu.make_async_copy(
                        attrw_hbm.at[wid * units_per_tile + u], a_buf, sem_g))

        def scatter_desc(u, m_buf):
            ub = lax.rem(lax.div(u, BATCH), 2)
            um = lax.rem(u, BATCH)
            return pltpu.make_async_copy(
                m_buf, acc_sh.at[dst_b.at[ub, um]], sem_s)

        load_batch(jnp.int32(0))
        for gd in gather_descs(jnp.int32(0), rows0, aw0):
            gd.start()

        def unit_body(u, r_cur, m_cur, a_cur, r_nxt, m_nxt, a_nxt):
            @pl.when(u >= 1)
            def _():
                scatter_desc(u - 1, m_nxt).wait()

            @pl.when(jnp.logical_and(lax.rem(u + 1, BATCH) == 0,
                                     u + 1 < units_per_tile))
            def _():
                load_batch(lax.div(u + 1, BATCH))

            @pl.when(u + 1 < units_per_tile)
            def _():
                for gd in gather_descs(u + 1, r_nxt, a_nxt):
                    gd.start()

            for gd in gather_descs(u, r_cur, a_cur):
                gd.wait()

            @pl.loop(0, G // 8)
            def _(cr):
                for cc in range(8):
                    c = cr * 8 + cc
                    e = a_cur[cr, pl.ds(cc * 16, 16)]
                    for k in range(D // 16):
                        a = r_cur[c, pl.ds(k * 16, 16)]
                        b = r_cur[c, pl.ds(D + k * 16, 16)]
                        m_cur[c, pl.ds(k * 16, 16)] = a + e * b

            scatter_desc(u, m_cur).start(add=True)

        @pl.loop(0, units_per_tile // 2)
        def _(t):
            unit_body(2 * t, rows0, msg0, aw0, rows1, msg1, aw1)
            unit_body(2 * t + 1, rows1, msg1, aw1, rows0, msg0, aw0)

        scatter_desc(jnp.int32(units_per_tile - 1), msg1).wait()
        plsc.subcore_barrier()

        @pl.loop(0, STRIPE // 128)
        def _(j):
            base = sid * STRIPE + j * 128
            pltpu.sync_copy(acc_sh.at[pl.ds(base, 128)],
                            out_hbm.at[cid, pl.ds(base, 128)])

    return sc_edge


def kernel(x, edge_index, edge_attr, W1, root1, b1, W2, root2, b2):
    E = edge_index.shape[1]
    upt = math.ceil(E / (NW * G * BATCH)) * BATCH
    e_pad = NW * G * upt

    src = jnp.pad(edge_index[0].astype(jnp.int32), (0, e_pad - E),
                  constant_values=N).reshape(-1, BATCH, G)
    dst = jnp.pad(edge_index[1].astype(jnp.int32), (0, e_pad - E),
                  constant_values=N).reshape(-1, BATCH, G)
    attr_flat = jnp.pad(edge_attr[:, 0], (0, e_pad - E))
    attrw = jnp.broadcast_to(attr_flat[:, None],
                             (e_pad, 16)).reshape(-1, G // 8, 128)

    cpt = e_pad // (NW * 128)
    dst128 = dst.reshape(-1, 128)

    sc_deg = _make_sc_deg(cpt)
    sc_edge = _make_sc_edge(upt)

    b1r = b1.reshape(1, D)
    b2r = b2.reshape(1, D)

    deg = sc_deg(dst128)
    d0 = deg[0].reshape(N_PAD, 1)[:N]
    d1 = deg[1].reshape(N_PAD, 1)[:N]

    tab1, xrb1 = _tc_tables(x, W1, root1, b1r)
    acc1 = sc_edge(tab1, src, dst, attrw)
    tab2, xrb2 = _tc_mid(acc1, d0, d1, xrb1, W2, root2, b2r)
    acc2 = sc_edge(tab2, src, dst, attrw)
    return _tc_out(acc2, d0, d1, xrb2)


# speedup vs baseline: 1.0032x; 1.0032x over previous
import dataclasses
import functools
import math

import numpy as np

import jax
import jax.numpy as jnp
from jax import lax
from jax.experimental import pallas as pl
from jax.experimental.pallas import tpu as pltpu
from jax.experimental.pallas import tpu_sc as plsc

N = 10000
D = 128
N_T = 10048
N_PAD = 10240
HR = N_PAD // 128
G = 56
BATCH = 8
NC = 2
NS = 16
NW = NC * NS
STRIPE = N_PAD // NS

PERM = np.empty((D,), np.int32)
for _k in range(D // 32):
    for _j in range(16):
        PERM[32 * _k + _j] = 32 * _k + 2 * _j
        PERM[32 * _k + 16 + _j] = 32 * _k + 2 * _j + 1
IPERM = np.argsort(PERM).astype(np.int32)


def _sc_compiler_params():
    cp = pltpu.CompilerParams()
    if "needs_layout_passes" in pltpu.CompilerParams.__dataclass_fields__:
        cp = dataclasses.replace(cp, needs_layout_passes=False)
    return cp


def _tc_tables_body(x_ref, w_ref, root_ref, b_ref, tab_ref, xrb_ref):
    x = x_ref[...]
    w0 = w_ref[0]
    wd = w_ref[1] - w_ref[0]
    tab_ref[:N, :D] = jnp.dot(
        x, w0, preferred_element_type=jnp.float32).astype(jnp.bfloat16)
    tab_ref[:N, D:] = jnp.dot(
        x, wd, preferred_element_type=jnp.float32).astype(jnp.bfloat16)
    tab_ref[N:, :] = jnp.zeros((N_T - N, 2 * D), jnp.bfloat16)
    xrb_ref[...] = (jnp.dot(x, root_ref[...], preferred_element_type=jnp.float32)
                    + b_ref[...])


def _combine(acc_ref, d0_ref, d1_ref, xrb_ref):
    num = acc_ref[0, :N, :] + acc_ref[1, :N, :]
    rec = 1.0 / jnp.clip(d0_ref[...] + d1_ref[...], 1.0, None)
    return num * rec + xrb_ref[...]


def _tc_mid_body(acc_ref, d0_ref, d1_ref, xrb_ref, w_ref, root_ref, b_ref,
                 tab_ref, xrb2_ref):
    h = _combine(acc_ref, d0_ref, d1_ref, xrb_ref)
    h = jnp.where(h > 0, h, jnp.exp(h) - 1.0)
    w0 = w_ref[0]
    wd = w_ref[1] - w_ref[0]
    tab_ref[:N, :D] = jnp.dot(
        h, w0, preferred_element_type=jnp.float32).astype(jnp.bfloat16)
    tab_ref[:N, D:] = jnp.dot(
        h, wd, preferred_element_type=jnp.float32).astype(jnp.bfloat16)
    tab_ref[N:, :] = jnp.zeros((N_T - N, 2 * D), jnp.bfloat16)
    xrb2_ref[...] = (jnp.dot(h, root_ref[...], preferred_element_type=jnp.float32)
                     + b_ref[...])


def _tc_out_body(acc_ref, d0_ref, d1_ref, xrb_ref, out_ref):
    z = _combine(acc_ref, d0_ref, d1_ref, xrb_ref)
    m = jnp.max(z, axis=1, keepdims=True)
    zz = z - m
    lse = jnp.log(jnp.sum(jnp.exp(zz), axis=1, keepdims=True))
    out_ref[...] = zz - lse


_tc_tables = pl.pallas_call(
    _tc_tables_body,
    out_shape=(jax.ShapeDtypeStruct((N_T, 2 * D), jnp.bfloat16),
               jax.ShapeDtypeStruct((N, D), jnp.float32)),
)

_tc_mid = pl.pallas_call(
    _tc_mid_body,
    out_shape=(jax.ShapeDtypeStruct((N_T, 2 * D), jnp.bfloat16),
               jax.ShapeDtypeStruct((N, D), jnp.float32)),
)

_tc_out = pl.pallas_call(
    _tc_out_body,
    out_shape=jax.ShapeDtypeStruct((N, D), jnp.float32),
)


def _make_sc_deg(chunks_per_tile):
    mesh = plsc.VectorSubcoreMesh(core_axis_name="c", subcore_axis_name="s")

    @functools.partial(
        pl.kernel,
        out_type=jax.ShapeDtypeStruct((NC, HR, 128), jnp.float32),
        mesh=mesh,
        compiler_params=_sc_compiler_params(),
        scratch_types=[
            pltpu.VMEM((1, 128), jnp.int32),
            pltpu.VMEM((HR, 128), jnp.float32),
            pltpu.VMEM((1, HR), jnp.int32),
            pltpu.VMEM_SHARED((HR, 128), jnp.float32),
        ],
    )
    def sc_deg(dst_hbm, out_hbm, dst_v, hist_v, idx_v, hist_sh):
        cid = lax.axis_index("c")
        sid = lax.axis_index("s")
        wid = cid * NS + sid

        @pl.loop(0, HR)
        def _(r):
            for k in range(128 // 16):
                hist_v[r, pl.ds(k * 16, 16)] = jnp.zeros((16,), jnp.float32)

        for k in range(HR // 16):
            idx_v[0, pl.ds(k * 16, 16)] = lax.iota(jnp.int32, 16) + (k * 16)

        @pl.when(sid < HR // 8)
        def _():
            pltpu.sync_copy(hist_v.at[pl.ds(0, 8)],
                            hist_sh.at[pl.ds(sid * 8, 8)])
        plsc.subcore_barrier()

        ones16 = jnp.ones((16,), jnp.float32)

        @pl.loop(0, chunks_per_tile)
        def _(j):
            row = wid * chunks_per_tile + j
            pltpu.sync_copy(dst_hbm.at[pl.ds(row, 1)], dst_v)
            for t in range(128 // 16):
                d = dst_v[0, pl.ds(t * 16, 16)]
                plsc.addupdate_scatter(
                    hist_v,
                    [lax.shift_right_logical(d, 7), lax.bitwise_and(d, 127)],
                    ones16)

        pltpu.sync_copy(hist_v, hist_sh.at[idx_v.at[0]], add=True)
        plsc.subcore_barrier()

        @pl.when(sid < HR // 8)
        def _():
            pltpu.sync_copy(hist_sh.at[pl.ds(sid * 8, 8)],
                            out_hbm.at[cid, pl.ds(sid * 8, 8)])

    return sc_deg


def _make_sc_edge(units_per_tile):
    assert units_per_tile % 2 == 0 and units_per_tile % BATCH == 0
    batches_per_tile = units_per_tile // BATCH
    mesh = plsc.VectorSubcoreMesh(core_axis_name="c", subcore_axis_name="s")

    @functools.partial(
        pl.kernel,
        out_type=jax.ShapeDtypeStruct((NC, N_PAD, D), jnp.float32),
        mesh=mesh,
        compiler_params=_sc_compiler_params(),
        scratch_types=[
            pltpu.VMEM((2, BATCH, G), jnp.int32),
            pltpu.VMEM((2, BATCH, G), jnp.int32),
            pltpu.VMEM((G // 8, 128), jnp.float32),
            pltpu.VMEM((G // 8, 128), jnp.float32),
            pltpu.VMEM((G, D), jnp.int32),
            pltpu.VMEM((G, D), jnp.int32),
            pltpu.VMEM((G, D), jnp.float32),
            pltpu.VMEM((G, D), jnp.float32),
            pltpu.VMEM_SHARED((N_PAD, D), jnp.float32),
            pltpu.SemaphoreType.DMA,
            pltpu.SemaphoreType.DMA,
        ],
    )
    def sc_edge(tab_hbm, src_hbm, dst_hbm, attrw_hbm, out_hbm,
                src_b, dst_b, aw0, aw1, rows0, rows1, msg0, msg1,
                acc_sh, sem_g, sem_s):
        cid = lax.axis_index("c")
        sid = lax.axis_index("s")
        wid = cid * NS + sid

        @pl.loop(0, G)
        def _(i):
            for k in range(D // 16):
                msg0[i, pl.ds(k * 16, 16)] = jnp.zeros((16,), jnp.float32)

        @pl.loop(0, STRIPE // 8)
        def _(j):
            pltpu.sync_copy(msg0.at[pl.ds(0, 8)],
                            acc_sh.at[pl.ds(sid * STRIPE + j * 8, 8)])

        plsc.subcore_barrier()

        def load_batch(b):
            row = wid * batches_per_tile + b
            slot = lax.rem(b, 2)
            pltpu.sync_copy(src_hbm.at[row], src_b.at[slot])
            pltpu.sync_copy(dst_hbm.at[row], dst_b.at[slot])

        def gather_descs(u, r_buf, a_buf):
            ub = lax.rem(lax.div(u, BATCH), 2)
            um = lax.rem(u, BATCH)
            idx = src_b.at[ub, um]
            return (pltpu.make_async_copy(tab_hbm.at[idx], r_buf, sem_g),
                    pltpu.make_async_copy(
                        attrw_hbm.at[wid * units_per_tile + u], a_buf, sem_g))

        def scatter_desc(u, m_buf):
            ub = lax.rem(lax.div(u, BATCH), 2)
            um = lax.rem(u, BATCH)
            return pltpu.make_async_copy(
                m_buf, acc_sh.at[dst_b.at[ub, um]], sem_s)

        load_batch(jnp.int32(0))
        for gd in gather_descs(jnp.int32(0), rows0, aw0):
            gd.start()

        def unit_body(u, r_cur, m_cur, a_cur, r_nxt, m_nxt, a_nxt):
            @pl.when(u >= 1)
            def _():
                scatter_desc(u - 1, m_nxt).wait()

            @pl.when(jnp.logical_and(lax.rem(u + 1, BATCH) == 0,
                                     u + 1 < units_per_tile))
            def _():
                load_batch(lax.div(u + 1, BATCH))

            @pl.when(u + 1 < units_per_tile)
            def _():
                for gd in gather_descs(u + 1, r_nxt, a_nxt):
                    gd.start()

            for gd in gather_descs(u, r_cur, a_cur):
                gd.wait()

            @pl.loop(0, G // 8)
            def _(cr):
                for cc in range(8):
                    c = cr * 8 + cc
                    e = a_cur[cr, pl.ds(cc * 16, 16)]
                    for k in range(D // 32):
                        up = plsc.bitcast(r_cur[c, pl.ds(k * 16, 16)],
                                          jnp.bfloat16)
                        vp = plsc.bitcast(r_cur[c, pl.ds(64 + k * 16, 16)],
                                          jnp.bfloat16)
                        ulo, uhi = plsc.unpack(
                            up, format=plsc.PackFormat.INTERLEAVED,
                            preferred_element_type=jnp.float32)
                        vlo, vhi = plsc.unpack(
                            vp, format=plsc.PackFormat.INTERLEAVED,
                            preferred_element_type=jnp.float32)
                        m_cur[c, pl.ds(k * 32, 16)] = ulo + e * vlo
                        m_cur[c, pl.ds(k * 32 + 16, 16)] = uhi + e * vhi

            scatter_desc(u, m_cur).start(add=True)

        @pl.loop(0, units_per_tile // 2)
        def _(t):
            unit_body(2 * t, rows0, msg0, aw0, rows1, msg1, aw1)
            unit_body(2 * t + 1, rows1, msg1, aw1, rows0, msg0, aw0)

        scatter_desc(jnp.int32(units_per_tile - 1), msg1).wait()
        plsc.subcore_barrier()

        @pl.loop(0, STRIPE // 128)
        def _(j):
            base = sid * STRIPE + j * 128
            pltpu.sync_copy(acc_sh.at[pl.ds(base, 128)],
                            out_hbm.at[cid, pl.ds(base, 128)])

    return sc_edge


def kernel(x, edge_index, edge_attr, W1, root1, b1, W2, root2, b2):
    E = edge_index.shape[1]
    upt = math.ceil(E / (NW * G * BATCH)) * BATCH
    e_pad = NW * G * upt

    src = jnp.pad(edge_index[0].astype(jnp.int32), (0, e_pad - E),
                  constant_values=N).reshape(-1, BATCH, G)
    dst = jnp.pad(edge_index[1].astype(jnp.int32), (0, e_pad - E),
                  constant_values=N).reshape(-1, BATCH, G)
    attr_flat = jnp.pad(edge_attr[:, 0], (0, e_pad - E))
    attrw = jnp.broadcast_to(attr_flat[:, None],
                             (e_pad, 16)).reshape(-1, G // 8, 128)

    cpt = e_pad // (NW * 128)
    dst128 = dst.reshape(-1, 128)

    sc_deg = _make_sc_deg(cpt)
    sc_edge = _make_sc_edge(upt)

    perm = jnp.asarray(PERM)
    root1p = root1[:, perm]
    b1r = b1[perm].reshape(1, D)
    W2p = W2[:, perm, :]
    root2p = root2[perm][:, perm]
    b2r = b2[perm].reshape(1, D)

    deg = sc_deg(dst128)
    d0 = deg[0].reshape(N_PAD, 1)[:N]
    d1 = deg[1].reshape(N_PAD, 1)[:N]

    tab1, xrb1 = _tc_tables(x, W1, root1p, b1r)
    tab1i = lax.bitcast_convert_type(tab1.reshape(N_T, D, 2), jnp.int32)
    acc1 = sc_edge(tab1i, src, dst, attrw)
    tab2, xrb2 = _tc_mid(acc1, d0, d1, xrb1, W2p, root2p, b2r)
    tab2i = lax.bitcast_convert_type(tab2.reshape(N_T, D, 2), jnp.int32)
    acc2 = sc_edge(tab2i, src, dst, attrw)
    out = _tc_out(acc2, d0, d1, xrb2)
    return out[:, jnp.asarray(IPERM)]


# speedup vs baseline: 1.0233x; 1.0201x over previous
import dataclasses
import functools
import math

import numpy as np

import jax
import jax.numpy as jnp
from jax import lax
from jax.experimental import pallas as pl
from jax.experimental.pallas import tpu as pltpu
from jax.experimental.pallas import tpu_sc as plsc

N = 10000
D = 128
N_T = 10048
N_PAD = 10240
HR = N_PAD // 128
G = 64
BATCH = 8
NC = 2
NS = 16
NW = NC * NS
STRIPE = N_PAD // NS

PERM = np.empty((D,), np.int32)
for _k in range(D // 32):
    for _j in range(16):
        PERM[32 * _k + _j] = 32 * _k + 2 * _j
        PERM[32 * _k + 16 + _j] = 32 * _k + 2 * _j + 1
IPERM = np.argsort(PERM).astype(np.int32)


def _sc_compiler_params():
    cp = pltpu.CompilerParams()
    if "needs_layout_passes" in pltpu.CompilerParams.__dataclass_fields__:
        cp = dataclasses.replace(cp, needs_layout_passes=False)
    return cp


def _tc_tables_body(x_ref, w_ref, root_ref, b_ref, tab_ref, xrb_ref):
    x = x_ref[...]
    w0 = w_ref[0]
    wd = w_ref[1] - w_ref[0]
    tab_ref[:N, :D] = jnp.dot(
        x, w0, preferred_element_type=jnp.float32).astype(jnp.bfloat16)
    tab_ref[:N, D:] = jnp.dot(
        x, wd, preferred_element_type=jnp.float32).astype(jnp.bfloat16)
    tab_ref[N:, :] = jnp.zeros((N_T - N, 2 * D), jnp.bfloat16)
    xrb_ref[...] = (jnp.dot(x, root_ref[...], preferred_element_type=jnp.float32)
                    + b_ref[...])


def _combine(acc_ref, d0_ref, d1_ref, xrb_ref):
    num = acc_ref[0, :N, :] + acc_ref[1, :N, :]
    rec = 1.0 / jnp.clip(d0_ref[...] + d1_ref[...], 1.0, None)
    return num * rec + xrb_ref[...]


def _tc_mid_body(acc_ref, d0_ref, d1_ref, xrb_ref, w_ref, root_ref, b_ref,
                 tab_ref, xrb2_ref):
    h = _combine(acc_ref, d0_ref, d1_ref, xrb_ref)
    h = jnp.where(h > 0, h, jnp.exp(h) - 1.0)
    w0 = w_ref[0]
    wd = w_ref[1] - w_ref[0]
    tab_ref[:N, :D] = jnp.dot(
        h, w0, preferred_element_type=jnp.float32).astype(jnp.bfloat16)
    tab_ref[:N, D:] = jnp.dot(
        h, wd, preferred_element_type=jnp.float32).astype(jnp.bfloat16)
    tab_ref[N:, :] = jnp.zeros((N_T - N, 2 * D), jnp.bfloat16)
    xrb2_ref[...] = (jnp.dot(h, root_ref[...], preferred_element_type=jnp.float32)
                     + b_ref[...])


def _tc_out_body(acc_ref, d0_ref, d1_ref, xrb_ref, out_ref):
    z = _combine(acc_ref, d0_ref, d1_ref, xrb_ref)
    m = jnp.max(z, axis=1, keepdims=True)
    zz = z - m
    lse = jnp.log(jnp.sum(jnp.exp(zz), axis=1, keepdims=True))
    out_ref[...] = zz - lse


_tc_tables = pl.pallas_call(
    _tc_tables_body,
    out_shape=(jax.ShapeDtypeStruct((N_T, 2 * D), jnp.bfloat16),
               jax.ShapeDtypeStruct((N, D), jnp.float32)),
)

_tc_mid = pl.pallas_call(
    _tc_mid_body,
    out_shape=(jax.ShapeDtypeStruct((N_T, 2 * D), jnp.bfloat16),
               jax.ShapeDtypeStruct((N, D), jnp.float32)),
)

_tc_out = pl.pallas_call(
    _tc_out_body,
    out_shape=jax.ShapeDtypeStruct((N, D), jnp.float32),
)


def _make_sc_deg(chunks_per_tile):
    mesh = plsc.VectorSubcoreMesh(core_axis_name="c", subcore_axis_name="s")

    @functools.partial(
        pl.kernel,
        out_type=jax.ShapeDtypeStruct((NC, HR, 128), jnp.float32),
        mesh=mesh,
        compiler_params=_sc_compiler_params(),
        scratch_types=[
            pltpu.VMEM((1, 128), jnp.int32),
            pltpu.VMEM((HR, 128), jnp.float32),
            pltpu.VMEM((1, HR), jnp.int32),
            pltpu.VMEM_SHARED((HR, 128), jnp.float32),
        ],
    )
    def sc_deg(dst_hbm, out_hbm, dst_v, hist_v, idx_v, hist_sh):
        cid = lax.axis_index("c")
        sid = lax.axis_index("s")
        wid = cid * NS + sid

        @pl.loop(0, HR)
        def _(r):
            for k in range(128 // 16):
                hist_v[r, pl.ds(k * 16, 16)] = jnp.zeros((16,), jnp.float32)

        for k in range(HR // 16):
            idx_v[0, pl.ds(k * 16, 16)] = lax.iota(jnp.int32, 16) + (k * 16)

        @pl.when(sid < HR // 8)
        def _():
            pltpu.sync_copy(hist_v.at[pl.ds(0, 8)],
                            hist_sh.at[pl.ds(sid * 8, 8)])
        plsc.subcore_barrier()

        ones16 = jnp.ones((16,), jnp.float32)

        @pl.loop(0, chunks_per_tile)
        def _(j):
            row = wid * chunks_per_tile + j
            pltpu.sync_copy(dst_hbm.at[pl.ds(row, 1)], dst_v)
            for t in range(128 // 16):
                d = dst_v[0, pl.ds(t * 16, 16)]
                plsc.addupdate_scatter(
                    hist_v,
                    [lax.shift_right_logical(d, 7), lax.bitwise_and(d, 127)],
                    ones16)

        pltpu.sync_copy(hist_v, hist_sh.at[idx_v.at[0]], add=True)
        plsc.subcore_barrier()

        @pl.when(sid < HR // 8)
        def _():
            pltpu.sync_copy(hist_sh.at[pl.ds(sid * 8, 8)],
                            out_hbm.at[cid, pl.ds(sid * 8, 8)])

    return sc_deg


def _make_sc_edge(units_per_tile):
    assert units_per_tile % 2 == 0 and units_per_tile % BATCH == 0
    batches_per_tile = units_per_tile // BATCH
    mesh = plsc.VectorSubcoreMesh(core_axis_name="c", subcore_axis_name="s")

    @functools.partial(
        pl.kernel,
        out_type=jax.ShapeDtypeStruct((NC, N_PAD, D), jnp.float32),
        mesh=mesh,
        compiler_params=_sc_compiler_params(),
        scratch_types=[
            pltpu.VMEM((2, BATCH, G), jnp.int32),
            pltpu.VMEM((2, BATCH, G), jnp.int32),
            pltpu.VMEM((G // 8, 128), jnp.float32),
            pltpu.VMEM((G // 8, 128), jnp.float32),
            pltpu.VMEM((G, D), jnp.int32),
            pltpu.VMEM((G, D), jnp.int32),
            pltpu.VMEM((G, D), jnp.float32),
            pltpu.VMEM((G, D), jnp.float32),
            pltpu.VMEM_SHARED((N_PAD, D), jnp.float32),
            pltpu.SemaphoreType.DMA,
            pltpu.SemaphoreType.DMA,
        ],
    )
    def sc_edge(tab_hbm, src_hbm, dst_hbm, attrw_hbm, out_hbm,
                src_b, dst_b, aw0, aw1, rows0, rows1, msg0, msg1,
                acc_sh, sem_g, sem_s):
        cid = lax.axis_index("c")
        sid = lax.axis_index("s")
        wid = cid * NS + sid

        @pl.loop(0, G)
        def _(i):
            for k in range(D // 16):
                msg0[i, pl.ds(k * 16, 16)] = jnp.zeros((16,), jnp.float32)

        @pl.loop(0, STRIPE // 8)
        def _(j):
            pltpu.sync_copy(msg0.at[pl.ds(0, 8)],
                            acc_sh.at[pl.ds(sid * STRIPE + j * 8, 8)])

        plsc.subcore_barrier()

        def load_batch(b):
            row = wid * batches_per_tile + b
            slot = lax.rem(b, 2)
            pltpu.sync_copy(src_hbm.at[row], src_b.at[slot])
            pltpu.sync_copy(dst_hbm.at[row], dst_b.at[slot])

        def gather_descs(u, r_buf, a_buf):
            ub = lax.rem(lax.div(u, BATCH), 2)
            um = lax.rem(u, BATCH)
            descs = [
                pltpu.make_async_copy(
                    tab_hbm.at[src_b.at[ub, um, pl.ds(q * 16, 16)]],
                    r_buf.at[pl.ds(q * 16, 16)], sem_g)
                for q in range(G // 16)
            ]
            descs.append(pltpu.make_async_copy(
                attrw_hbm.at[wid * units_per_tile + u], a_buf, sem_g))
            return descs

        def scatter_desc(u, m_buf):
            ub = lax.rem(lax.div(u, BATCH), 2)
            um = lax.rem(u, BATCH)
            return pltpu.make_async_copy(
                m_buf, acc_sh.at[dst_b.at[ub, um]], sem_s)

        load_batch(jnp.int32(0))
        for gd in gather_descs(jnp.int32(0), rows0, aw0):
            gd.start()

        def unit_body(u, r_cur, m_cur, a_cur, r_nxt, m_nxt, a_nxt):
            @pl.when(u >= 1)
            def _():
                scatter_desc(u - 1, m_nxt).wait()

            @pl.when(jnp.logical_and(lax.rem(u + 1, BATCH) == 0,
                                     u + 1 < units_per_tile))
            def _():
                load_batch(lax.div(u + 1, BATCH))

            @pl.when(u + 1 < units_per_tile)
            def _():
                for gd in gather_descs(u + 1, r_nxt, a_nxt):
                    gd.start()

            for gd in gather_descs(u, r_cur, a_cur):
                gd.wait()

            @pl.loop(0, G // 8)
            def _(cr):
                for cc in range(8):
                    c = cr * 8 + cc
                    e = a_cur[cr, pl.ds(cc * 16, 16)]
                    for k in range(D // 32):
                        up = plsc.bitcast(r_cur[c, pl.ds(k * 16, 16)],
                                          jnp.bfloat16)
                        vp = plsc.bitcast(r_cur[c, pl.ds(64 + k * 16, 16)],
                                          jnp.bfloat16)
                        ulo, uhi = plsc.unpack(
                            up, format=plsc.PackFormat.INTERLEAVED,
                            preferred_element_type=jnp.float32)
                        vlo, vhi = plsc.unpack(
                            vp, format=plsc.PackFormat.INTERLEAVED,
                            preferred_element_type=jnp.float32)
                        m_cur[c, pl.ds(k * 32, 16)] = ulo + e * vlo
                        m_cur[c, pl.ds(k * 32 + 16, 16)] = uhi + e * vhi

            scatter_desc(u, m_cur).start(add=True)

        @pl.loop(0, units_per_tile // 2)
        def _(t):
            unit_body(2 * t, rows0, msg0, aw0, rows1, msg1, aw1)
            unit_body(2 * t + 1, rows1, msg1, aw1, rows0, msg0, aw0)

        scatter_desc(jnp.int32(units_per_tile - 1), msg1).wait()
        plsc.subcore_barrier()

        @pl.loop(0, STRIPE // 128)
        def _(j):
            base = sid * STRIPE + j * 128
            pltpu.sync_copy(acc_sh.at[pl.ds(base, 128)],
                            out_hbm.at[cid, pl.ds(base, 128)])

    return sc_edge


def kernel(x, edge_index, edge_attr, W1, root1, b1, W2, root2, b2):
    E = edge_index.shape[1]
    upt = math.ceil(E / (NW * G * BATCH)) * BATCH
    e_pad = NW * G * upt

    src = jnp.pad(edge_index[0].astype(jnp.int32), (0, e_pad - E),
                  constant_values=N).reshape(-1, BATCH, G)
    dst = jnp.pad(edge_index[1].astype(jnp.int32), (0, e_pad - E),
                  constant_values=N).reshape(-1, BATCH, G)
    attr_flat = jnp.pad(edge_attr[:, 0], (0, e_pad - E))
    attrw = jnp.broadcast_to(attr_flat[:, None],
                             (e_pad, 16)).reshape(-1, G // 8, 128)

    cpt = e_pad // (NW * 128)
    dst128 = dst.reshape(-1, 128)

    sc_deg = _make_sc_deg(cpt)
    sc_edge = _make_sc_edge(upt)

    perm = jnp.asarray(PERM)
    root1p = root1[:, perm]
    b1r = b1[perm].reshape(1, D)
    W2p = W2[:, perm, :]
    root2p = root2[perm][:, perm]
    b2r = b2[perm].reshape(1, D)

    deg = sc_deg(dst128)
    d0 = deg[0].reshape(N_PAD, 1)[:N]
    d1 = deg[1].reshape(N_PAD, 1)[:N]

    tab1, xrb1 = _tc_tables(x, W1, root1p, b1r)
    tab1i = lax.bitcast_convert_type(tab1.reshape(N_T, D, 2), jnp.int32)
    acc1 = sc_edge(tab1i, src, dst, attrw)
    tab2, xrb2 = _tc_mid(acc1, d0, d1, xrb1, W2p, root2p, b2r)
    tab2i = lax.bitcast_convert_type(tab2.reshape(N_T, D, 2), jnp.int32)
    acc2 = sc_edge(tab2i, src, dst, attrw)
    out = _tc_out(acc2, d0, d1, xrb2)
    return out[:, jnp.asarray(IPERM)]


# speedup vs baseline: 1.0530x; 1.0290x over previous
import dataclasses
import functools
import math

import jax
import jax.numpy as jnp
from jax import lax
from jax.experimental import pallas as pl
from jax.experimental.pallas import tpu as pltpu
from jax.experimental.pallas import tpu_sc as plsc

N = 10000
D = 128
N_T = 10048
N_PAD = 10240
HR = N_PAD // 128
G = 64
BATCH = 16
NC = 2
NS = 16
NW = NC * NS
STRIPE = N_PAD // NS


def _sc_compiler_params():
    cp = pltpu.CompilerParams()
    if "needs_layout_passes" in pltpu.CompilerParams.__dataclass_fields__:
        cp = dataclasses.replace(cp, needs_layout_passes=False)
    return cp


def _tc_tables_body(x_ref, w_ref, root_ref, b_ref, tu_ref, tv_ref, xrb_ref):
    x = x_ref[...]
    w0 = w_ref[0]
    wd = w_ref[1] - w_ref[0]
    tu_ref[:N, :] = jnp.dot(x, w0, preferred_element_type=jnp.float32)
    tv_ref[:N, :] = jnp.dot(x, wd, preferred_element_type=jnp.float32)
    tu_ref[N:, :] = jnp.zeros((N_T - N, D), jnp.float32)
    tv_ref[N:, :] = jnp.zeros((N_T - N, D), jnp.float32)
    xrb_ref[...] = (jnp.dot(x, root_ref[...], preferred_element_type=jnp.float32)
                    + b_ref[...])


def _combine(acc_ref, d0_ref, d1_ref, xrb_ref):
    num = acc_ref[0, :N, :] + acc_ref[1, :N, :]
    rec = 1.0 / jnp.clip(d0_ref[...] + d1_ref[...], 1.0, None)
    return num * rec + xrb_ref[...]


def _tc_mid_body(acc_ref, d0_ref, d1_ref, xrb_ref, w_ref, root_ref, b_ref,
                 tu_ref, tv_ref, xrb2_ref):
    h = _combine(acc_ref, d0_ref, d1_ref, xrb_ref)
    h = jnp.where(h > 0, h, jnp.exp(h) - 1.0)
    w0 = w_ref[0]
    wd = w_ref[1] - w_ref[0]
    tu_ref[:N, :] = jnp.dot(h, w0, preferred_element_type=jnp.float32)
    tv_ref[:N, :] = jnp.dot(h, wd, preferred_element_type=jnp.float32)
    tu_ref[N:, :] = jnp.zeros((N_T - N, D), jnp.float32)
    tv_ref[N:, :] = jnp.zeros((N_T - N, D), jnp.float32)
    xrb2_ref[...] = (jnp.dot(h, root_ref[...], preferred_element_type=jnp.float32)
                     + b_ref[...])


def _tc_out_body(acc_ref, d0_ref, d1_ref, xrb_ref, out_ref):
    z = _combine(acc_ref, d0_ref, d1_ref, xrb_ref)
    m = jnp.max(z, axis=1, keepdims=True)
    zz = z - m
    lse = jnp.log(jnp.sum(jnp.exp(zz), axis=1, keepdims=True))
    out_ref[...] = zz - lse


_tc_tables = pl.pallas_call(
    _tc_tables_body,
    out_shape=(jax.ShapeDtypeStruct((N_T, D), jnp.float32),
               jax.ShapeDtypeStruct((N_T, D), jnp.float32),
               jax.ShapeDtypeStruct((N, D), jnp.float32)),
)

_tc_mid = pl.pallas_call(
    _tc_mid_body,
    out_shape=(jax.ShapeDtypeStruct((N_T, D), jnp.float32),
               jax.ShapeDtypeStruct((N_T, D), jnp.float32),
               jax.ShapeDtypeStruct((N, D), jnp.float32)),
)

_tc_out = pl.pallas_call(
    _tc_out_body,
    out_shape=jax.ShapeDtypeStruct((N, D), jnp.float32),
)


def _make_sc_deg(chunks_per_tile):
    mesh = plsc.VectorSubcoreMesh(core_axis_name="c", subcore_axis_name="s")

    @functools.partial(
        pl.kernel,
        out_type=jax.ShapeDtypeStruct((NC, HR, 128), jnp.float32),
        mesh=mesh,
        compiler_params=_sc_compiler_params(),
        scratch_types=[
            pltpu.VMEM((1, 128), jnp.int32),
            pltpu.VMEM((HR, 128), jnp.float32),
            pltpu.VMEM((1, HR), jnp.int32),
            pltpu.VMEM_SHARED((HR, 128), jnp.float32),
        ],
    )
    def sc_deg(dst_hbm, out_hbm, dst_v, hist_v, idx_v, hist_sh):
        cid = lax.axis_index("c")
        sid = lax.axis_index("s")
        wid = cid * NS + sid

        @pl.loop(0, HR)
        def _(r):
            for k in range(128 // 16):
                hist_v[r, pl.ds(k * 16, 16)] = jnp.zeros((16,), jnp.float32)

        for k in range(HR // 16):
            idx_v[0, pl.ds(k * 16, 16)] = lax.iota(jnp.int32, 16) + (k * 16)

        @pl.when(sid < HR // 8)
        def _():
            pltpu.sync_copy(hist_v.at[pl.ds(0, 8)],
                            hist_sh.at[pl.ds(sid * 8, 8)])
        plsc.subcore_barrier()

        ones16 = jnp.ones((16,), jnp.float32)

        @pl.loop(0, chunks_per_tile)
        def _(j):
            row = wid * chunks_per_tile + j
            pltpu.sync_copy(dst_hbm.at[pl.ds(row, 1)], dst_v)
            for t in range(128 // 16):
                d = dst_v[0, pl.ds(t * 16, 16)]
                plsc.addupdate_scatter(
                    hist_v,
                    [lax.shift_right_logical(d, 7), lax.bitwise_and(d, 127)],
                    ones16)

        pltpu.sync_copy(hist_v, hist_sh.at[idx_v.at[0]], add=True)
        plsc.subcore_barrier()

        @pl.when(sid < HR // 8)
        def _():
            pltpu.sync_copy(hist_sh.at[pl.ds(sid * 8, 8)],
                            out_hbm.at[cid, pl.ds(sid * 8, 8)])

    return sc_deg


def _make_sc_edge(units_per_tile):
    assert units_per_tile % 2 == 0 and units_per_tile % BATCH == 0
    batches_per_tile = units_per_tile // BATCH
    mesh = plsc.VectorSubcoreMesh(core_axis_name="c", subcore_axis_name="s")

    @functools.partial(
        pl.kernel,
        out_type=jax.ShapeDtypeStruct((NC, N_PAD, D), jnp.float32),
        mesh=mesh,
        compiler_params=_sc_compiler_params(),
        scratch_types=[
            pltpu.VMEM((2, BATCH, G), jnp.int32),
            pltpu.VMEM((2, BATCH, G), jnp.int32),
            pltpu.VMEM((G // 8, 128), jnp.float32),
            pltpu.VMEM((G // 8, 128), jnp.float32),
            pltpu.VMEM((G, D), jnp.float32),
            pltpu.VMEM((G, D), jnp.float32),
            pltpu.VMEM((G, D), jnp.float32),
            pltpu.VMEM((G, D), jnp.float32),
            pltpu.VMEM_SHARED((N_PAD, D), jnp.float32),
            pltpu.SemaphoreType.DMA,
            pltpu.SemaphoreType.DMA,
        ],
    )
    def sc_edge(tu_hbm, tv_hbm, src_hbm, dst_hbm, attrw_hbm, out_hbm,
                src_b, dst_b, aw0, aw1, urows0, urows1, vrows0, vrows1,
                acc_sh, sem_g, sem_s):
        cid = lax.axis_index("c")
        sid = lax.axis_index("s")
        wid = cid * NS + sid

        @pl.loop(0, G)
        def _(i):
            for k in range(D // 16):
                urows0[i, pl.ds(k * 16, 16)] = jnp.zeros((16,), jnp.float32)

        @pl.loop(0, STRIPE // G)
        def _(j):
            pltpu.sync_copy(urows0,
                            acc_sh.at[pl.ds(sid * STRIPE + j * G, G)])

        plsc.subcore_barrier()

        def load_batch(b):
            row = wid * batches_per_tile + b
            slot = lax.rem(b, 2)
            pltpu.sync_copy(src_hbm.at[row], src_b.at[slot])
            pltpu.sync_copy(dst_hbm.at[row], dst_b.at[slot])

        def gather_descs(u, u_buf, v_buf, a_buf):
            ub = lax.rem(lax.div(u, BATCH), 2)
            um = lax.rem(u, BATCH)
            idx = src_b.at[ub, um]
            return (pltpu.make_async_copy(tu_hbm.at[idx], u_buf, sem_g),
                    pltpu.make_async_copy(tv_hbm.at[idx], v_buf, sem_g),
                    pltpu.make_async_copy(
                        attrw_hbm.at[wid * units_per_tile + u], a_buf, sem_g))

        def scatter_desc(u, u_buf):
            ub = lax.rem(lax.div(u, BATCH), 2)
            um = lax.rem(u, BATCH)
            return pltpu.make_async_copy(
                u_buf, acc_sh.at[dst_b.at[ub, um]], sem_s)

        load_batch(jnp.int32(0))
        for gd in gather_descs(jnp.int32(0), urows0, vrows0, aw0):
            gd.start()

        def unit_body(u, u_cur, v_cur, a_cur, u_nxt, v_nxt, a_nxt):
            @pl.when(u >= 1)
            def _():
                scatter_desc(u - 1, u_nxt).wait()

            @pl.when(jnp.logical_and(lax.rem(u + 1, BATCH) == 0,
                                     u + 1 < units_per_tile))
            def _():
                load_batch(lax.div(u + 1, BATCH))

            @pl.when(u + 1 < units_per_tile)
            def _():
                for gd in gather_descs(u + 1, u_nxt, v_nxt, a_nxt):
                    gd.start()

            for gd in gather_descs(u, u_cur, v_cur, a_cur):
                gd.wait()

            @pl.loop(0, G // 8)
            def _(cr):
                for cc in range(8):
                    c = cr * 8 + cc
                    e = a_cur[cr, pl.ds(cc * 16, 16)]
                    for k in range(D // 16):
                        a = u_cur[c, pl.ds(k * 16, 16)]
                        b = v_cur[c, pl.ds(k * 16, 16)]
                        u_cur[c, pl.ds(k * 16, 16)] = a + e * b

            scatter_desc(u, u_cur).start(add=True)

        @pl.loop(0, units_per_tile // 2)
        def _(t):
            unit_body(2 * t, urows0, vrows0, aw0, urows1, vrows1, aw1)
            unit_body(2 * t + 1, urows1, vrows1, aw1, urows0, vrows0, aw0)

        scatter_desc(jnp.int32(units_per_tile - 1), urows1).wait()
        plsc.subcore_barrier()

        @pl.loop(0, STRIPE // 128)
        def _(j):
            base = sid * STRIPE + j * 128
            pltpu.sync_copy(acc_sh.at[pl.ds(base, 128)],
                            out_hbm.at[cid, pl.ds(base, 128)])

    return sc_edge


def kernel(x, edge_index, edge_attr, W1, root1, b1, W2, root2, b2):
    E = edge_index.shape[1]
    upt = math.ceil(E / (NW * G * BATCH)) * BATCH
    e_pad = NW * G * upt

    src = jnp.pad(edge_index[0].astype(jnp.int32), (0, e_pad - E),
                  constant_values=N).reshape(-1, BATCH, G)
    dst = jnp.pad(edge_index[1].astype(jnp.int32), (0, e_pad - E),
                  constant_values=N).reshape(-1, BATCH, G)
    attr_flat = jnp.pad(edge_attr[:, 0], (0, e_pad - E))
    attrw = jnp.broadcast_to(attr_flat[:, None],
                             (e_pad, 16)).reshape(-1, G // 8, 128)

    cpt = e_pad // (NW * 128)
    dst128 = dst.reshape(-1, 128)

    sc_deg = _make_sc_deg(cpt)
    sc_edge = _make_sc_edge(upt)

    b1r = b1.reshape(1, D)
    b2r = b2.reshape(1, D)

    deg = sc_deg(dst128)
    d0 = deg[0].reshape(N_PAD, 1)[:N]
    d1 = deg[1].reshape(N_PAD, 1)[:N]

    tu1, tv1, xrb1 = _tc_tables(x, W1, root1, b1r)
    acc1 = sc_edge(tu1, tv1, src, dst, attrw)
    tu2, tv2, xrb2 = _tc_mid(acc1, d0, d1, xrb1, W2, root2, b2r)
    acc2 = sc_edge(tu2, tv2, src, dst, attrw)
    return _tc_out(acc2, d0, d1, xrb2)
